# Initial kernel scaffold; baseline (speedup 1.0000x reference)
import jax
import jax.numpy as jnp
from jax.experimental import pallas as pl


def kernel(x, t_emb, Wd0, We0, W):
    raise NotImplementedError("write your pallas kernel here")


# speedup vs baseline: 1.7830x; 1.7830x over previous
import jax
import jax.numpy as jnp
from jax import lax
from jax.experimental import pallas as pl
from jax.experimental.pallas import tpu as pltpu

_B, _N, _T = 4, 512, 2048
_F = _T // 2 + 1
_H = 64
_E = 24
_K = 32


def _min_max(t):
    mn = jnp.min(t, axis=-1, keepdims=True)
    mx = jnp.max(t, axis=-1, keepdims=True)
    return (t - mn) / (mx - mn + 1.0)


def _l2_normalize(t):
    n = jnp.linalg.norm(t, ord=2, axis=2, keepdims=True)
    return t / jnp.maximum(n, 1e-12)


def _proj_body(xn_ref, tn_ref, wd_ref, we_ref, h_ref):
    bf = jnp.bfloat16
    e = jnp.dot(xn_ref[...].astype(bf), wd_ref[...].astype(bf),
                preferred_element_type=jnp.float32)
    cat = jnp.concatenate([e, tn_ref[...]], axis=-1)
    h = jnp.dot(cat.astype(bf), we_ref[...].astype(bf),
                preferred_element_type=jnp.float32)
    h_ref[...] = jnp.maximum(h, 0.0)


def _adj_body(loc_ref, w_ref, out_ref):
    bf = jnp.bfloat16
    loc = loc_ref[0]
    locT = loc.T
    wb32 = w_ref[...].astype(bf).astype(jnp.float32)
    adj = jnp.zeros((_N, _N), jnp.float32)
    for i in range(_H):
        col = loc[:, i:i + 1]
        row = locT[i:i + 1, :]
        p = (col * row).astype(bf).astype(jnp.float32)
        adj = adj + p * wb32[0:1, i:i + 1]
    adj = jnp.maximum(adj, 0.0)

    ai = lax.bitcast_convert_type(adj, jnp.int32)
    lo = jnp.zeros((_N, 1), jnp.int32)
    for bit in range(30, -1, -1):
        cand = lo | jnp.int32(1 << bit)
        cnt = jnp.sum((ai >= cand).astype(jnp.int32), axis=-1, keepdims=True)
        lo = jnp.where(cnt >= _K, cand, lo)
    th = lax.bitcast_convert_type(lo, jnp.float32)
    zt = jnp.where(adj >= th, adj, 0.0)
    out_ref[0] = zt / (jnp.sum(zt, axis=-1, keepdims=True) + 1e-5)


def kernel(x, t_emb, Wd0, We0, W):
    xa = jnp.abs(jnp.fft.rfft(x, axis=-1, norm='ortho'))
    xn = _l2_normalize(_min_max(xa)).reshape(_B * _N, _F)
    tn = _l2_normalize(_min_max(t_emb)).reshape(_B * _N, _E)

    h = pl.pallas_call(
        _proj_body,
        in_specs=[
            pl.BlockSpec((_B * _N, _F), lambda: (0, 0)),
            pl.BlockSpec((_B * _N, _E), lambda: (0, 0)),
            pl.BlockSpec((_F, _H), lambda: (0, 0)),
            pl.BlockSpec((_H + _E, _H), lambda: (0, 0)),
        ],
        out_specs=pl.BlockSpec((_B * _N, _H), lambda: (0, 0)),
        out_shape=jax.ShapeDtypeStruct((_B * _N, _H), jnp.float32),
    )(xn, tn, Wd0, We0).reshape(_B, _N, _H)

    m = jnp.mean(h, axis=-1, keepdims=True)
    v = jnp.var(h, axis=-1, keepdims=True)
    loc = (h - m) / jnp.sqrt(v + 1e-8)

    return pl.pallas_call(
        _adj_body,
        grid=(_B,),
        in_specs=[
            pl.BlockSpec((1, _N, _H), lambda b: (b, 0, 0)),
            pl.BlockSpec((1, _H), lambda b: (0, 0)),
        ],
        out_specs=pl.BlockSpec((1, _N, _N), lambda b: (b, 0, 0)),
        out_shape=jax.ShapeDtypeStruct((_B, _N, _N), jnp.float32),
        compiler_params=pltpu.CompilerParams(
            dimension_semantics=("arbitrary",),
        ),
    )(loc, W.reshape(1, _H))
